# Initial kernel scaffold; baseline (speedup 1.0000x reference)
import jax
import jax.numpy as jnp
from jax.experimental import pallas as pl


def kernel(userID, itemID, rels, mode, user_emb, item_emb):
    raise NotImplementedError("write your pallas kernel here")


# speedup vs baseline: 1.4690x; 1.4690x over previous
import functools

import jax
import jax.numpy as jnp
from jax import lax
from jax.experimental import pallas as pl
from jax.experimental.pallas import tpu as pltpu
from jax.experimental.pallas import tpu_sc as plsc

D = 32
L = 16
CHUNK = 128


def _build_sc_call(B, n_workers):
    n_per_w = B // n_workers
    n_chunks = n_per_w // CHUNK
    mesh = plsc.VectorSubcoreMesh(core_axis_name="c", subcore_axis_name="s")
    num_cores = mesh.num_cores

    @functools.partial(
        pl.kernel,
        out_type=jax.ShapeDtypeStruct((B,), jnp.float32),
        mesh=mesh,
        compiler_params=pltpu.CompilerParams(
            needs_layout_passes=False, use_tc_tiling_on_sc=False),
        scratch_types=[
            pltpu.VMEM((CHUNK,), jnp.int32),
            pltpu.VMEM((CHUNK,), jnp.int32),
            pltpu.VMEM((CHUNK, D), jnp.float32),
            pltpu.VMEM((CHUNK, D), jnp.float32),
            pltpu.VMEM((CHUNK,), jnp.float32),
            pltpu.SemaphoreType.DMA,
        ],
    )
    def sc_call(uid_hbm, iid_hbm, uemb_hbm, iemb_hbm, out_hbm,
                uidx_v, iidx_v, urows_v, irows_v, out_v, sem):
        wid = lax.axis_index("s") * num_cores + lax.axis_index("c")
        base = wid * n_per_w
        lane = lax.iota(jnp.int32, L)

        def chunk_body(t):
            off = base + t * CHUNK
            pltpu.sync_copy(uid_hbm.at[pl.ds(off, CHUNK)], uidx_v)
            pltpu.sync_copy(iid_hbm.at[pl.ds(off, CHUNK)], iidx_v)
            cu = pltpu.async_copy(uemb_hbm.at[uidx_v], urows_v, sem)
            ci = pltpu.async_copy(iemb_hbm.at[iidx_v], irows_v, sem)
            cu.wait()
            ci.wait()

            lane0 = lane == 0

            def out_body(b):
                u0 = urows_v[b, pl.ds(0, L)]
                u1 = urows_v[b, pl.ds(L, L)]
                v0 = irows_v[b, pl.ds(0, L)]
                v1 = irows_v[b, pl.ds(L, L)]
                s = jnp.sum(u0 * v0 + u1 * v1)
                plsc.store_scatter(
                    out_v,
                    [jnp.full((L,), b, jnp.int32)],
                    jnp.full((L,), s, jnp.float32),
                    mask=lane0,
                )

            pl.loop(0, CHUNK)(out_body)
            pltpu.sync_copy(out_v, out_hbm.at[pl.ds(off, CHUNK)])

        pl.loop(0, n_chunks)(chunk_body)

    return sc_call


@jax.jit
def _listmf(uid, iid, user_emb, item_emb):
    B = uid.shape[0]
    sc_call = _build_sc_call(B, 32)
    return sc_call(uid, iid, user_emb, item_emb)


def kernel(userID, itemID, rels, mode, user_emb, item_emb):
    shape = userID.shape
    uid = jnp.asarray(userID, jnp.int32).reshape(-1)
    iid = jnp.asarray(itemID, jnp.int32).reshape(-1)
    out = _listmf(uid, iid, user_emb, item_emb)
    return out.reshape(shape)


# speedup vs baseline: 2.4933x; 1.6973x over previous
import functools

import jax
import jax.numpy as jnp
from jax import lax
from jax.experimental import pallas as pl
from jax.experimental.pallas import tpu as pltpu
from jax.experimental.pallas import tpu_sc as plsc

D = 32
L = 16
IW = 128
SUB = 4
CH = SUB * IW
N_WORKERS = 32


def _build_sc_call(B):
    n_per_w = B // N_WORKERS
    n_iters = n_per_w // CH
    n_rows_w = n_per_w // IW
    mesh = plsc.VectorSubcoreMesh(core_axis_name="c", subcore_axis_name="s")
    num_cores = mesh.num_cores

    @functools.partial(
        pl.kernel,
        out_type=jax.ShapeDtypeStruct((B,), jnp.float32),
        mesh=mesh,
        compiler_params=pltpu.CompilerParams(
            needs_layout_passes=False, use_tc_tiling_on_sc=False),
        scratch_types=[
            [pltpu.VMEM((SUB, IW), jnp.int32) for _ in range(2)],
            [pltpu.VMEM((SUB, IW), jnp.int32) for _ in range(2)],
            [pltpu.VMEM((CH, D), jnp.float32) for _ in range(2)],
            [pltpu.VMEM((CH, D), jnp.float32) for _ in range(2)],
            [pltpu.VMEM((CH,), jnp.float32) for _ in range(2)],
            pltpu.SemaphoreType.DMA,
            pltpu.SemaphoreType.DMA,
            pltpu.SemaphoreType.DMA,
        ],
    )
    def sc_call(uid_hbm, iid_hbm, uemb_hbm, iemb_hbm, out_hbm,
                uidx, iidx, urows, irows, outb, sem_idx, sem_rows, sem_out):
        wid = lax.axis_index("s") * num_cores + lax.axis_index("c")
        wrow = wid * n_rows_w
        base = wid * n_per_w
        lane = lax.iota(jnp.int32, L)
        lane0 = lane == 0

        def fire_idx(s, p):
            goff = wrow + s * SUB
            pltpu.async_copy(uid_hbm.at[pl.ds(goff, SUB)], uidx[p], sem_idx)
            pltpu.async_copy(iid_hbm.at[pl.ds(goff, SUB)], iidx[p], sem_idx)

        def wait_idx():
            pltpu.make_async_copy(
                uid_hbm.at[pl.ds(0, SUB)], uidx[0], sem_idx).wait()
            pltpu.make_async_copy(
                iid_hbm.at[pl.ds(0, SUB)], iidx[0], sem_idx).wait()

        def fire_rows(p):
            for r in range(SUB):
                pltpu.async_copy(
                    uemb_hbm.at[uidx[p].at[r]],
                    urows[p].at[pl.ds(r * IW, IW)], sem_rows)
                pltpu.async_copy(
                    iemb_hbm.at[iidx[p].at[r]],
                    irows[p].at[pl.ds(r * IW, IW)], sem_rows)

        def wait_rows():
            pltpu.make_async_copy(
                uemb_hbm.at[pl.ds(0, CH)], urows[0], sem_rows).wait()
            pltpu.make_async_copy(
                iemb_hbm.at[pl.ds(0, CH)], irows[0], sem_rows).wait()

        def wait_out():
            pltpu.make_async_copy(
                outb[0], out_hbm.at[pl.ds(0, CH)], sem_out).wait()

        fire_idx(0, 0)
        fire_idx(1, 1)
        wait_idx()
        fire_rows(0)

        def iter_body(s, par):
            nxt = 1 - par
            wait_rows()

            @pl.when(s + 1 < n_iters)
            def _():
                wait_idx()
                fire_rows(nxt)

            @pl.when(s + 2 < n_iters)
            def _():
                fire_idx(s + 2, par)

            @pl.when(s >= 2)
            def _():
                wait_out()

            ur = urows[par]
            ir = irows[par]
            ob = outb[par]

            def out_body(b):
                u0 = ur[b, pl.ds(0, L)]
                u1 = ur[b, pl.ds(L, L)]
                v0 = ir[b, pl.ds(0, L)]
                v1 = ir[b, pl.ds(L, L)]
                s_ = jnp.sum(u0 * v0 + u1 * v1)
                plsc.store_scatter(
                    ob,
                    [jnp.full((L,), b, jnp.int32)],
                    jnp.full((L,), s_, jnp.float32),
                    mask=lane0,
                )

            plsc.parallel_loop(0, CH, unroll=8)(out_body)
            pltpu.async_copy(ob, out_hbm.at[pl.ds(base + s * CH, CH)], sem_out)

        def pair_body(t):
            iter_body(2 * t, 0)
            iter_body(2 * t + 1, 1)

        pl.loop(0, n_iters // 2)(pair_body)

        wait_out()
        wait_out()

    return sc_call


@jax.jit
def _listmf(uid, iid, user_emb, item_emb):
    B = uid.size
    sc_call = _build_sc_call(B)
    return sc_call(uid.reshape(B // IW, IW), iid.reshape(B // IW, IW),
                   user_emb, item_emb)


def kernel(userID, itemID, rels, mode, user_emb, item_emb):
    shape = userID.shape
    uid = jnp.asarray(userID, jnp.int32).reshape(-1)
    iid = jnp.asarray(itemID, jnp.int32).reshape(-1)
    out = _listmf(uid, iid, user_emb, item_emb)
    return out.reshape(shape)
